# Initial kernel scaffold; baseline (speedup 1.0000x reference)
import jax
import jax.numpy as jnp
from jax.experimental import pallas as pl


def kernel(x, edge_index, edge_attr, W1, b1, W2, b2):
    raise NotImplementedError("write your pallas kernel here")


# speedup vs baseline: 22.5825x; 22.5825x over previous
import functools

import jax
import jax.numpy as jnp
from jax import lax
from jax.experimental import pallas as pl
from jax.experimental.pallas import tpu as pltpu
from jax.experimental.pallas import tpu_sc as plsc

N = 10000
E = 320000
D = 128
H = 64
C = 40
CP = 48
NC = 2
NS = 16
L = 16
NW = NC * NS
EPW = E // NW
CH = 80
NCH = EPW // CH
NP = 10240
RPT = NP // NS

_mesh = plsc.VectorSubcoreMesh(
    core_axis_name="c", subcore_axis_name="s", num_cores=NC, num_subcores=NS)


@functools.partial(
    pl.kernel,
    out_type=jax.ShapeDtypeStruct((NC, N), jnp.float32),
    mesh=_mesh,
    scratch_types=[
        pltpu.VMEM((NCH, CH), jnp.int32),
        pltpu.VMEM((NCH, CH), jnp.float32),
        pltpu.VMEM((N,), jnp.float32),
        pltpu.VMEM_SHARED((N,), jnp.float32),
    ],
)
def _sc_degree(dst_hbm, ew_hbm, out_hbm, dst_v, ew_v, zb_v, acc_sh):
    c = lax.axis_index("c")
    s = lax.axis_index("s")
    wid = s * NC + c
    pltpu.sync_copy(dst_hbm.at[wid], dst_v)
    pltpu.sync_copy(ew_hbm.at[wid], ew_v)

    @pl.when(s == 0)
    def _zero():
        zeros = jnp.zeros((L,), jnp.float32)

        def zb(i, carry):
            zb_v[pl.ds(i * L, L)] = zeros
            return carry

        lax.fori_loop(0, N // L, zb, 0)
        pltpu.sync_copy(zb_v, acc_sh)

    plsc.subcore_barrier()

    def chunk(j, carry):
        pltpu.sync_copy(ew_v.at[j], acc_sh.at[dst_v.at[j]], add=True)
        return carry

    lax.fori_loop(0, NCH, chunk, 0)
    plsc.subcore_barrier()

    @pl.when(s == 0)
    def _out():
        pltpu.sync_copy(acc_sh, out_hbm.at[c])


def _make_sc_agg(F):
    @functools.partial(
        pl.kernel,
        out_type=jax.ShapeDtypeStruct((NC, NP, F), jnp.float32),
        mesh=_mesh,
        compiler_params=pltpu.CompilerParams(use_tc_tiling_on_sc=False),
        scratch_types=[
            pltpu.VMEM((NCH, CH), jnp.int32),
            pltpu.VMEM((NCH, CH), jnp.int32),
            pltpu.VMEM((NCH, CH), jnp.float32),
            pltpu.VMEM((CH, F), jnp.float32),
            pltpu.VMEM((RPT, F), jnp.float32),
            pltpu.VMEM_SHARED((NP, F), jnp.float32),
            pltpu.SemaphoreType.DMA,
        ],
    )
    def sc_agg(g_hbm, src_hbm, dst_hbm, ew_hbm, out_hbm,
               src_v, dst_v, ew_v, rows_v, zb_v, acc_sh, sem):
        c = lax.axis_index("c")
        s = lax.axis_index("s")
        wid = s * NC + c
        pltpu.sync_copy(src_hbm.at[wid], src_v)
        pltpu.sync_copy(dst_hbm.at[wid], dst_v)
        pltpu.sync_copy(ew_hbm.at[wid], ew_v)

        zeros = jnp.zeros((L,), jnp.float32)

        def zb(i, carry):
            for g in range(F // L):
                zb_v[i, pl.ds(g * L, L)] = zeros
            return carry

        lax.fori_loop(0, RPT, zb, 0)
        pltpu.sync_copy(zb_v, acc_sh.at[pl.ds(s * RPT, RPT)])
        plsc.subcore_barrier()

        def chunk(j, carry):
            pltpu.async_copy(g_hbm.at[src_v.at[j]], rows_v, sem).wait()
            for g16 in range(CH // L):
                wvec = ew_v[j, pl.ds(g16 * L, L)]
                for i in range(L):
                    e = g16 * L + i
                    w = jnp.full((L,), wvec[i])
                    for g in range(F // L):
                        sl = pl.ds(g * L, L)
                        rows_v[e, sl] = rows_v[e, sl] * w
            pltpu.sync_copy(rows_v, acc_sh.at[dst_v.at[j]], add=True)
            return carry

        lax.fori_loop(0, NCH, chunk, 0)
        plsc.subcore_barrier()
        pltpu.sync_copy(acc_sh.at[pl.ds(s * RPT, RPT)],
                        out_hbm.at[c, pl.ds(s * RPT, RPT)])

    return sc_agg


_sc_agg_h = _make_sc_agg(H)
_sc_agg_c = _make_sc_agg(CP)


def _tc1_body(x_ref, w1_ref, degp_ref, h1_ref, g1_ref, dinv_ref):
    deg = degp_ref[0, :] + degp_ref[1, :] + 1.0
    dinv = jnp.where(deg > 0,
                     lax.rsqrt(jnp.maximum(deg, 1e-12)),
                     jnp.zeros_like(deg))
    h1 = jnp.dot(x_ref[:, :], w1_ref[:, :],
                 preferred_element_type=jnp.float32)
    h1_ref[:, :] = h1
    g1_ref[:, :] = h1 * dinv[:, None]
    dinv_ref[:, :] = dinv[:, None]


def _tc2_body(accp_ref, g1_ref, dinv_ref, b1_ref, w2p_ref, g2_ref):
    dinv = dinv_ref[:, :]
    a1 = (accp_ref[0, :N, :] + accp_ref[1, :N, :] + g1_ref[:, :] * dinv) * dinv
    z = jnp.maximum(a1 + b1_ref[:, :], 0.0)
    h2 = jnp.dot(z, w2p_ref[:, :], preferred_element_type=jnp.float32)
    g2_ref[:, :] = h2 * dinv


def _tc3_body(accp_ref, g2_ref, dinv_ref, b2_ref, out_ref):
    dinv = dinv_ref[:, :]
    a2full = (accp_ref[0, :N, :] + accp_ref[1, :N, :] + g2_ref[:, :] * dinv) * dinv
    a2 = a2full[:, :C] + b2_ref[:, :]
    m = jnp.max(a2, axis=1, keepdims=True)
    sh = a2 - m
    lse = jnp.log(jnp.sum(jnp.exp(sh), axis=1, keepdims=True))
    out_ref[:, :] = sh - lse


_tc1 = pl.pallas_call(
    _tc1_body,
    out_shape=(
        jax.ShapeDtypeStruct((N, H), jnp.float32),
        jax.ShapeDtypeStruct((N, H), jnp.float32),
        jax.ShapeDtypeStruct((N, 1), jnp.float32),
    ),
)

_tc2 = pl.pallas_call(
    _tc2_body,
    out_shape=jax.ShapeDtypeStruct((N, CP), jnp.float32),
)

_tc3 = pl.pallas_call(
    _tc3_body,
    out_shape=jax.ShapeDtypeStruct((N, C), jnp.float32),
)


def kernel(x, edge_index, edge_attr, W1, b1, W2, b2):
    src = edge_index[0].astype(jnp.int32).reshape(NW, NCH, CH)
    dst = edge_index[1].astype(jnp.int32).reshape(NW, NCH, CH)
    ew = edge_attr.astype(jnp.float32).reshape(NW, NCH, CH)

    degp = _sc_degree(dst, ew)
    h1, g1, dinv = _tc1(x, W1, degp)
    acc1 = _sc_agg_h(g1, src, dst, ew)
    w2p = jnp.concatenate(
        [W2, jnp.zeros((H, CP - C), jnp.float32)], axis=1)
    g2 = _tc2(acc1, g1, dinv, b1.reshape(1, H), w2p)
    acc2 = _sc_agg_c(g2, src, dst, ew)
    out = _tc3(acc2, g2, dinv, b2.reshape(1, C))
    return out
